# baseline (device time: 13727 ns/iter reference)
import jax
import jax.numpy as jnp
from jax import lax
from jax.experimental import pallas as pl
from jax.experimental.pallas import tpu as pltpu

N_DEV = 4
N_TOK = 256
D_IN = 128
D_OUT = 256
N_EXP = 8
EXP_PER_DEV = 2
ROWS_PER_DEV = N_TOK // N_DEV


def kernel(x, router_W, route_idx, expert_W):
    def body(x_ref, rw_ref, idx_ref, ew_ref, out_ref,
             acc_ref, send_ref, recv_ref, send_sems, recv_sems):
        my = lax.axis_index("i")
        left = jnp.mod(my - 1, N_DEV)
        right = jnp.mod(my + 1, N_DEV)

        barrier_sem = pltpu.get_barrier_semaphore()
        for nbr in (left, right):
            pl.semaphore_signal(
                barrier_sem, inc=1,
                device_id=(nbr,), device_id_type=pl.DeviceIdType.MESH,
            )
        pl.semaphore_wait(barrier_sem, 2)

        xv = x_ref[:, :]
        scores = jnp.dot(xv, rw_ref[:, :],
                         preferred_element_type=jnp.float32)
        m = jnp.max(scores, axis=-1, keepdims=True)
        p = jnp.exp(scores - m)
        p = p / jnp.sum(p, axis=-1, keepdims=True)
        iota8 = lax.broadcasted_iota(jnp.int32, (N_TOK, N_EXP), 1)
        mask = (iota8 == idx_ref[:, 0:1]) | (iota8 == idx_ref[:, 1:2])
        w = jnp.where(mask, p, 0.0)
        w = w / jnp.sum(w, axis=-1, keepdims=True)

        acc = jnp.zeros((N_TOK, D_OUT), jnp.float32)
        for j in range(EXP_PER_DEV):
            eg = my * EXP_PER_DEV + j
            wcol = jnp.sum(jnp.where(iota8 == eg, w, 0.0),
                           axis=1, keepdims=True)
            xs = (wcol * xv).astype(jnp.bfloat16)
            Wj = ew_ref[j].astype(jnp.bfloat16)
            acc = acc + jnp.dot(xs, Wj,
                                preferred_element_type=jnp.float32)
        acc_ref[:, :] = acc

        c0 = jnp.mod(my - 1, N_DEV)
        send_ref[0, :, :] = acc_ref[pl.ds(c0 * ROWS_PER_DEV, ROWS_PER_DEV), :]
        for s in range(N_DEV - 1):
            rdma = pltpu.make_async_remote_copy(
                src_ref=send_ref.at[s],
                dst_ref=recv_ref.at[s],
                send_sem=send_sems.at[s],
                recv_sem=recv_sems.at[s],
                device_id=(right,),
                device_id_type=pl.DeviceIdType.MESH,
            )
            rdma.start()
            rdma.wait()
            c_recv = jnp.mod(my - 2 - s, N_DEV)
            acc_chunk = acc_ref[pl.ds(c_recv * ROWS_PER_DEV, ROWS_PER_DEV), :]
            if s < N_DEV - 2:
                send_ref[s + 1, :, :] = recv_ref[s, :, :] + acc_chunk
            else:
                out_ref[:, :] = recv_ref[s, :, :] + acc_chunk

    return pl.pallas_call(
        body,
        out_shape=jax.ShapeDtypeStruct((ROWS_PER_DEV, D_OUT), jnp.float32),
        in_specs=[
            pl.BlockSpec(memory_space=pltpu.VMEM),
            pl.BlockSpec(memory_space=pltpu.VMEM),
            pl.BlockSpec(memory_space=pltpu.VMEM),
            pl.BlockSpec(memory_space=pltpu.VMEM),
        ],
        out_specs=pl.BlockSpec(memory_space=pltpu.VMEM),
        scratch_shapes=[
            pltpu.VMEM((N_TOK, D_OUT), jnp.float32),
            pltpu.VMEM((N_DEV - 1, ROWS_PER_DEV, D_OUT), jnp.float32),
            pltpu.VMEM((N_DEV - 1, ROWS_PER_DEV, D_OUT), jnp.float32),
            pltpu.SemaphoreType.DMA((N_DEV - 1,)),
            pltpu.SemaphoreType.DMA((N_DEV - 1,)),
        ],
        compiler_params=pltpu.CompilerParams(collective_id=0),
    )(x, router_W, route_idx, expert_W)


# device time: 8383 ns/iter; 1.6375x vs baseline; 1.6375x over previous
import jax
import jax.numpy as jnp
from jax import lax
from jax.experimental import pallas as pl
from jax.experimental.pallas import tpu as pltpu

N_DEV = 4
N_TOK = 256
D_IN = 128
D_OUT = 256
N_EXP = 8
EXP_PER_DEV = 2
ROWS_PER_DEV = N_TOK // N_DEV


def kernel(x, router_W, route_idx, expert_W):
    def body(x_ref, rw_ref, idx_ref, ew_ref, out_ref,
             acc_ref, send_ref, recv_ref, send_sems, recv_sems):
        my = lax.axis_index("i")

        barrier_sem = pltpu.get_barrier_semaphore()
        for k in range(1, N_DEV):
            pl.semaphore_signal(
                barrier_sem, inc=1,
                device_id=(jnp.mod(my + k, N_DEV),),
                device_id_type=pl.DeviceIdType.MESH,
            )
        pl.semaphore_wait(barrier_sem, N_DEV - 1)

        xv = x_ref[:, :]
        scores = jnp.dot(xv, rw_ref[:, :],
                         preferred_element_type=jnp.float32)
        m = jnp.max(scores, axis=-1, keepdims=True)
        p = jnp.exp(scores - m)
        p = p / jnp.sum(p, axis=-1, keepdims=True)
        iota8 = lax.broadcasted_iota(jnp.int32, (N_TOK, N_EXP), 1)
        mask = (iota8 == idx_ref[:, 0:1]) | (iota8 == idx_ref[:, 1:2])
        w = jnp.where(mask, p, 0.0)
        w = w / jnp.sum(w, axis=-1, keepdims=True)

        acc = jnp.zeros((N_TOK, D_OUT), jnp.float32)
        for j in range(EXP_PER_DEV):
            eg = my * EXP_PER_DEV + j
            wcol = jnp.sum(jnp.where(iota8 == eg, w, 0.0),
                           axis=1, keepdims=True)
            xs = (wcol * xv).astype(jnp.bfloat16)
            Wj = ew_ref[j].astype(jnp.bfloat16)
            acc = acc + jnp.dot(xs, Wj,
                                preferred_element_type=jnp.float32)
        acc_ref[:, :] = acc

        rdmas = []
        for k in (2, 1, 3):
            dest = jnp.mod(my + k, N_DEV)
            send_ref[k - 1, :, :] = acc_ref[
                pl.ds(dest * ROWS_PER_DEV, ROWS_PER_DEV), :
            ].astype(jnp.bfloat16)
            rdma = pltpu.make_async_remote_copy(
                src_ref=send_ref.at[k - 1],
                dst_ref=recv_ref.at[k - 1],
                send_sem=send_sems.at[k - 1],
                recv_sem=recv_sems.at[k - 1],
                device_id=(dest,),
                device_id_type=pl.DeviceIdType.MESH,
            )
            rdma.start()
            rdmas.append(rdma)

        result = acc_ref[pl.ds(my * ROWS_PER_DEV, ROWS_PER_DEV), :]
        for i, rdma in enumerate(rdmas):
            rdma.wait_recv()
            k = (2, 1, 3)[i]
            result = result + recv_ref[k - 1, :, :].astype(jnp.float32)
        out_ref[:, :] = result
        for rdma in rdmas:
            rdma.wait_send()

    return pl.pallas_call(
        body,
        out_shape=jax.ShapeDtypeStruct((ROWS_PER_DEV, D_OUT), jnp.float32),
        in_specs=[
            pl.BlockSpec(memory_space=pltpu.VMEM),
            pl.BlockSpec(memory_space=pltpu.VMEM),
            pl.BlockSpec(memory_space=pltpu.VMEM),
            pl.BlockSpec(memory_space=pltpu.VMEM),
        ],
        out_specs=pl.BlockSpec(memory_space=pltpu.VMEM),
        scratch_shapes=[
            pltpu.VMEM((N_TOK, D_OUT), jnp.float32),
            pltpu.VMEM((N_DEV - 1, ROWS_PER_DEV, D_OUT), jnp.bfloat16),
            pltpu.VMEM((N_DEV - 1, ROWS_PER_DEV, D_OUT), jnp.bfloat16),
            pltpu.SemaphoreType.DMA((N_DEV - 1,)),
            pltpu.SemaphoreType.DMA((N_DEV - 1,)),
        ],
        compiler_params=pltpu.CompilerParams(collective_id=0),
    )(x, router_W, route_idx, expert_W)


# device time: 8220 ns/iter; 1.6700x vs baseline; 1.0198x over previous
import jax
import jax.numpy as jnp
from jax import lax
from jax.experimental import pallas as pl
from jax.experimental.pallas import tpu as pltpu

N_DEV = 4
N_TOK = 256
D_IN = 128
D_OUT = 256
N_EXP = 8
EXP_PER_DEV = 2
ROWS_PER_DEV = N_TOK // N_DEV


def kernel(x, router_W, route_idx, expert_W):
    def body(x_ref, rw_ref, idx_ref, ew_ref, out_ref,
             accb_ref, recv_ref, send_sems, recv_sems):
        my = lax.axis_index("i")

        barrier_sem = pltpu.get_barrier_semaphore()
        for k in range(1, N_DEV):
            pl.semaphore_signal(
                barrier_sem, inc=1,
                device_id=(jnp.mod(my + k, N_DEV),),
                device_id_type=pl.DeviceIdType.MESH,
            )

        xv = x_ref[:, :]
        scores = jnp.dot(xv, rw_ref[:, :],
                         preferred_element_type=jnp.float32)
        m = jnp.max(scores, axis=-1, keepdims=True)
        p = jnp.exp(scores - m)
        p = p / jnp.sum(p, axis=-1, keepdims=True)
        iota8 = lax.broadcasted_iota(jnp.int32, (N_TOK, N_EXP), 1)
        mask = (iota8 == idx_ref[:, 0:1]) | (iota8 == idx_ref[:, 1:2])
        w = jnp.where(mask, p, 0.0)
        w = w / jnp.sum(w, axis=-1, keepdims=True)

        w0 = jnp.sum(jnp.where(iota8 == my * EXP_PER_DEV, w, 0.0),
                     axis=1, keepdims=True)
        w1 = jnp.sum(jnp.where(iota8 == my * EXP_PER_DEV + 1, w, 0.0),
                     axis=1, keepdims=True)
        lhs = jnp.concatenate([w0 * xv, w1 * xv],
                              axis=1).astype(jnp.bfloat16)
        Wcat = ew_ref[:, :, :].reshape(
            EXP_PER_DEV * D_IN, D_OUT).astype(jnp.bfloat16)
        acc = jnp.dot(lhs, Wcat,
                      preferred_element_type=jnp.float32)

        for c in range(N_DEV):
            accb_ref[c, :, :] = acc[
                c * ROWS_PER_DEV:(c + 1) * ROWS_PER_DEV, :
            ].astype(jnp.bfloat16)

        pl.semaphore_wait(barrier_sem, N_DEV - 1)

        rdmas = {}
        for k in (2, 1, 3):
            dest = jnp.mod(my + k, N_DEV)
            rdma = pltpu.make_async_remote_copy(
                src_ref=accb_ref.at[dest],
                dst_ref=recv_ref.at[k - 1],
                send_sem=send_sems.at[k - 1],
                recv_sem=recv_sems.at[k - 1],
                device_id=(dest,),
                device_id_type=pl.DeviceIdType.MESH,
            )
            rdma.start()
            rdmas[k] = rdma

        result = accb_ref[my, :, :].astype(jnp.float32)
        for k in (1, 3, 2):
            rdmas[k].wait_recv()
            result = result + recv_ref[k - 1, :, :].astype(jnp.float32)
        out_ref[:, :] = result
        for k in (2, 1, 3):
            rdmas[k].wait_send()

    return pl.pallas_call(
        body,
        out_shape=jax.ShapeDtypeStruct((ROWS_PER_DEV, D_OUT), jnp.float32),
        in_specs=[
            pl.BlockSpec(memory_space=pltpu.VMEM),
            pl.BlockSpec(memory_space=pltpu.VMEM),
            pl.BlockSpec(memory_space=pltpu.VMEM),
            pl.BlockSpec(memory_space=pltpu.VMEM),
        ],
        out_specs=pl.BlockSpec(memory_space=pltpu.VMEM),
        scratch_shapes=[
            pltpu.VMEM((N_DEV, ROWS_PER_DEV, D_OUT), jnp.bfloat16),
            pltpu.VMEM((N_DEV - 1, ROWS_PER_DEV, D_OUT), jnp.bfloat16),
            pltpu.SemaphoreType.DMA((N_DEV - 1,)),
            pltpu.SemaphoreType.DMA((N_DEV - 1,)),
        ],
        compiler_params=pltpu.CompilerParams(collective_id=0),
    )(x, router_W, route_idx, expert_W)
